# Initial kernel scaffold; baseline (speedup 1.0000x reference)
import jax
import jax.numpy as jnp
from jax.experimental import pallas as pl


def kernel(x, edge_index, batch, Wr, br, Wd, bd, W1, b1, W2, b2, W3, b3, Wm1, bm1, Wm2, bm2):
    raise NotImplementedError("write your pallas kernel here")


# speedup vs baseline: 4.7284x; 4.7284x over previous
import jax
import jax.numpy as jnp
from jax import lax
from jax.experimental import pallas as pl

_N = 10000
_E = 320000
_H = 256
_G = 5000
_DF = 128
_RD = 6
_HALF = _N // 2
_HPAD = 5120
_NP = 2 * _HPAD
_GHALF = _G // 2
_GPAD = 2560
_GP = 2 * _GPAD
_R = 256


def _emb_body(x_ref, deg_ref, wr_ref, br_ref, wd_ref, bd_ref, w1_ref,
              s1_ref, dinv_ref):
    dinv = lax.rsqrt(deg_ref[...] + 1.0)
    xb = x_ref[...]
    embr = jnp.dot(xb, wr_ref[...], preferred_element_type=jnp.float32) + br_ref[...]
    embd = jnp.dot(xb, wd_ref[...], preferred_element_type=jnp.float32) + bd_ref[...]
    rows = pl.program_id(0) * _R + lax.broadcasted_iota(jnp.int32, (_R, 1), 0)
    emb = jnp.where(rows % 2 == 0, embr, embd)
    s1_ref[...] = jnp.dot(emb, w1_ref[...], preferred_element_type=jnp.float32) * dinv
    dinv_ref[...] = dinv


def _layer_body(acc_ref, s_ref, dinv_ref, b_ref, w_ref, out_ref):
    dinv = dinv_ref[...]
    h = jnp.maximum(dinv * (acc_ref[...] + s_ref[...]) + b_ref[...], 0.0)
    out_ref[...] = jnp.dot(h, w_ref[...], preferred_element_type=jnp.float32) * dinv


def _comb_body(acc_ref, s_ref, dinv_ref, b_ref, out_ref):
    dinv = dinv_ref[...]
    out_ref[...] = jnp.maximum(dinv * (acc_ref[...] + s_ref[...]) + b_ref[...], 0.0)


def _mlp_body(p_ref, wm1_ref, bm1_ref, wm2_ref, bm2_ref, out_ref):
    hidden = jnp.maximum(
        jnp.dot(p_ref[...], wm1_ref[...], preferred_element_type=jnp.float32)
        + bm1_ref[...], 0.0)
    out_ref[...] = jnp.sum(hidden * wm2_ref[...], axis=1, keepdims=True) + bm2_ref[...]


def _row_spec(width):
    return pl.BlockSpec((_R, width), lambda b: (b, 0))


def _full_spec(r, ccol):
    return pl.BlockSpec((r, ccol), lambda b: (0, 0))


_GRID_N = _NP // _R
_GRID_G = _GP // _R

_emb_call = pl.pallas_call(
    _emb_body,
    grid=(_GRID_N,),
    in_specs=[
        _row_spec(_DF),
        _row_spec(1),
        _full_spec(_DF, _H),
        _full_spec(1, _H),
        _full_spec(_DF, _H),
        _full_spec(1, _H),
        _full_spec(_H, _H),
    ],
    out_specs=[_row_spec(_H), _row_spec(1)],
    out_shape=[
        jax.ShapeDtypeStruct((_NP, _H), jnp.float32),
        jax.ShapeDtypeStruct((_NP, 1), jnp.float32),
    ],
)

_layer_call = pl.pallas_call(
    _layer_body,
    grid=(_GRID_N,),
    in_specs=[
        _row_spec(_H),
        _row_spec(_H),
        _row_spec(1),
        _full_spec(1, _H),
        _full_spec(_H, _H),
    ],
    out_specs=_row_spec(_H),
    out_shape=jax.ShapeDtypeStruct((_NP, _H), jnp.float32),
)

_comb_call = pl.pallas_call(
    _comb_body,
    grid=(_GRID_N,),
    in_specs=[
        _row_spec(_H),
        _row_spec(_H),
        _row_spec(1),
        _full_spec(1, _H),
    ],
    out_specs=_row_spec(_H),
    out_shape=jax.ShapeDtypeStruct((_NP, _H), jnp.float32),
)

_mlp_call = pl.pallas_call(
    _mlp_body,
    grid=(_GRID_G,),
    in_specs=[
        _row_spec(_H),
        _full_spec(_H, _H // 2),
        _full_spec(1, _H // 2),
        _full_spec(1, _H // 2),
        _full_spec(1, 1),
    ],
    out_specs=_row_spec(1),
    out_shape=jax.ShapeDtypeStruct((_GP, 1), jnp.float32),
)


def _scatter(s, srcp, dgp):
    return jnp.zeros((_NP, _H), jnp.float32).at[dgp].add(s[srcp])


def kernel(x, edge_index, batch, Wr, br, Wd, bd, W1, b1, W2, b2, W3, b3,
           Wm1, bm1, Wm2, bm2):
    src = edge_index[0]
    dst = edge_index[1]
    gap = _HPAD - _HALF
    srcp = src + gap * (src >= _HALF).astype(jnp.int32)
    dgp = dst + gap * (dst >= _HALF).astype(jnp.int32)
    x_p = jnp.concatenate([
        x[:_HALF], jnp.zeros((gap, _DF), jnp.float32),
        x[_HALF:], jnp.zeros((gap, _DF), jnp.float32)])
    ggap = _GPAD - _GHALF
    blg_pad = jnp.full((gap,), _GP - 1, jnp.int32)
    blg = jnp.concatenate([
        jnp.where(batch[:_HALF] < _GHALF, batch[:_HALF],
                  batch[:_HALF] + ggap), blg_pad,
        jnp.where(batch[_HALF:] < _GHALF, batch[_HALF:],
                  batch[_HALF:] + ggap), blg_pad])
    wrp = jnp.pad(Wr, ((0, _DF - _RD), (0, 0)))
    br2 = br.reshape(1, _H)
    bd2 = bd.reshape(1, _H)
    b1_2 = b1.reshape(1, _H)
    b2_2 = b2.reshape(1, _H)
    b3_2 = b3.reshape(1, _H)
    bm1_2 = bm1.reshape(1, _H // 2)
    wm2r = Wm2.reshape(1, _H // 2)
    bm2_2 = bm2.reshape(1, 1)

    deg2d = jnp.zeros((_NP,), jnp.float32).at[dgp].add(1.0)[:, None]
    s1, dinv2d = _emb_call(x_p, deg2d, wrp, br2, Wd, bd2, W1)
    acc1 = _scatter(s1, srcp, dgp)
    s2 = _layer_call(acc1, s1, dinv2d, b1_2, W2)
    acc2 = _scatter(s2, srcp, dgp)
    s3 = _layer_call(acc2, s2, dinv2d, b2_2, W3)
    acc3 = _scatter(s3, srcp, dgp)
    h3p = _comb_call(acc3, s3, dinv2d, b3_2)
    pooled = jnp.zeros((_GP, _H), jnp.float32).at[blg].add(h3p)
    pred_p = _mlp_call(pooled, Wm1, bm1_2, wm2r, bm2_2)
    return jnp.concatenate([pred_p[:_GHALF], pred_p[_GPAD:_GPAD + _GHALF]])


# speedup vs baseline: 5.0327x; 1.0644x over previous
import jax
import jax.numpy as jnp
from jax import lax
from jax.experimental import pallas as pl

_N = 10000
_E = 320000
_H = 256
_G = 5000
_DF = 128
_RD = 6
_HALF = _N // 2
_HPAD = 5120
_NP = 2 * _HPAD
_GHALF = _G // 2
_GPAD = 2560
_GP = 2 * _GPAD
_R = 256


def _emb_body(x_ref, deg_ref, wr_ref, br_ref, wd_ref, bd_ref, w1_ref,
              s1_ref, dinv_ref):
    dinv = lax.rsqrt(deg_ref[...] + 1.0)
    xb = x_ref[...]
    embr = jnp.dot(xb, wr_ref[...], preferred_element_type=jnp.float32) + br_ref[...]
    embd = jnp.dot(xb, wd_ref[...], preferred_element_type=jnp.float32) + bd_ref[...]
    rows = pl.program_id(0) * _R + lax.broadcasted_iota(jnp.int32, (_R, 1), 0)
    emb = jnp.where(rows % 2 == 0, embr, embd)
    s1_ref[...] = jnp.dot(emb, w1_ref[...], preferred_element_type=jnp.float32) * dinv
    dinv_ref[...] = dinv


def _layer_body(acc_ref, s_ref, dinv_ref, b_ref, w_ref, out_ref):
    dinv = dinv_ref[...]
    h = jnp.maximum(dinv * (acc_ref[...] + s_ref[...]) + b_ref[...], 0.0)
    out_ref[...] = jnp.dot(h, w_ref[...], preferred_element_type=jnp.float32) * dinv


def _comb_body(acc_ref, s_ref, dinv_ref, b_ref, out_ref):
    dinv = dinv_ref[...]
    out_ref[...] = jnp.maximum(dinv * (acc_ref[...] + s_ref[...]) + b_ref[...], 0.0)


def _mlp_body(p_ref, wm1_ref, bm1_ref, wm2_ref, bm2_ref, out_ref):
    hidden = jnp.maximum(
        jnp.dot(p_ref[...], wm1_ref[...], preferred_element_type=jnp.float32)
        + bm1_ref[...], 0.0)
    out_ref[...] = jnp.sum(hidden * wm2_ref[...], axis=1, keepdims=True) + bm2_ref[...]


def _row_spec(width):
    return pl.BlockSpec((_R, width), lambda b: (b, 0))


def _full_spec(r, ccol):
    return pl.BlockSpec((r, ccol), lambda b: (0, 0))


_GRID_N = _NP // _R
_GRID_G = _GP // _R

_emb_call = pl.pallas_call(
    _emb_body,
    grid=(_GRID_N,),
    in_specs=[
        _row_spec(_DF),
        _row_spec(1),
        _full_spec(_DF, _H),
        _full_spec(1, _H),
        _full_spec(_DF, _H),
        _full_spec(1, _H),
        _full_spec(_H, _H),
    ],
    out_specs=[_row_spec(_H), _row_spec(1)],
    out_shape=[
        jax.ShapeDtypeStruct((_NP, _H), jnp.float32),
        jax.ShapeDtypeStruct((_NP, 1), jnp.float32),
    ],
)

_layer_call = pl.pallas_call(
    _layer_body,
    grid=(_GRID_N,),
    in_specs=[
        _row_spec(_H),
        _row_spec(_H),
        _row_spec(1),
        _full_spec(1, _H),
        _full_spec(_H, _H),
    ],
    out_specs=_row_spec(_H),
    out_shape=jax.ShapeDtypeStruct((_NP, _H), jnp.float32),
)

_comb_call = pl.pallas_call(
    _comb_body,
    grid=(_GRID_N,),
    in_specs=[
        _row_spec(_H),
        _row_spec(_H),
        _row_spec(1),
        _full_spec(1, _H),
    ],
    out_specs=_row_spec(_H),
    out_shape=jax.ShapeDtypeStruct((_NP, _H), jnp.float32),
)

_mlp_call = pl.pallas_call(
    _mlp_body,
    grid=(_GRID_G,),
    in_specs=[
        _row_spec(_H),
        _full_spec(_H, _H // 2),
        _full_spec(1, _H // 2),
        _full_spec(1, _H // 2),
        _full_spec(1, 1),
    ],
    out_specs=_row_spec(1),
    out_shape=jax.ShapeDtypeStruct((_GP, 1), jnp.float32),
)


def _scatter(s, srcp_s, dgp_s):
    return jnp.zeros((_NP, _H), jnp.float32).at[dgp_s].add(
        s[srcp_s], indices_are_sorted=True)


def kernel(x, edge_index, batch, Wr, br, Wd, bd, W1, b1, W2, b2, W3, b3,
           Wm1, bm1, Wm2, bm2):
    src = edge_index[0]
    dst = edge_index[1]
    gap = _HPAD - _HALF
    srcp = src + gap * (src >= _HALF).astype(jnp.int32)
    dgp = dst + gap * (dst >= _HALF).astype(jnp.int32)
    x_p = jnp.concatenate([
        x[:_HALF], jnp.zeros((gap, _DF), jnp.float32),
        x[_HALF:], jnp.zeros((gap, _DF), jnp.float32)])
    ggap = _GPAD - _GHALF
    blg_pad = jnp.full((gap,), _GP - 1, jnp.int32)
    blg = jnp.concatenate([
        jnp.where(batch[:_HALF] < _GHALF, batch[:_HALF],
                  batch[:_HALF] + ggap), blg_pad,
        jnp.where(batch[_HALF:] < _GHALF, batch[_HALF:],
                  batch[_HALF:] + ggap), blg_pad])
    wrp = jnp.pad(Wr, ((0, _DF - _RD), (0, 0)))
    br2 = br.reshape(1, _H)
    bd2 = bd.reshape(1, _H)
    b1_2 = b1.reshape(1, _H)
    b2_2 = b2.reshape(1, _H)
    b3_2 = b3.reshape(1, _H)
    bm1_2 = bm1.reshape(1, _H // 2)
    wm2r = Wm2.reshape(1, _H // 2)
    bm2_2 = bm2.reshape(1, 1)

    order = jnp.argsort(dgp)
    dgp_s = dgp[order]
    srcp_s = srcp[order]
    deg2d = jnp.zeros((_NP,), jnp.float32).at[dgp_s].add(
        1.0, indices_are_sorted=True)[:, None]
    s1, dinv2d = _emb_call(x_p, deg2d, wrp, br2, Wd, bd2, W1)
    acc1 = _scatter(s1, srcp_s, dgp_s)
    s2 = _layer_call(acc1, s1, dinv2d, b1_2, W2)
    acc2 = _scatter(s2, srcp_s, dgp_s)
    s3 = _layer_call(acc2, s2, dinv2d, b2_2, W3)
    acc3 = _scatter(s3, srcp_s, dgp_s)
    h3p = _comb_call(acc3, s3, dinv2d, b3_2)
    pooled = jnp.zeros((_GP, _H), jnp.float32).at[blg].add(h3p)
    pred_p = _mlp_call(pooled, Wm1, bm1_2, wm2r, bm2_2)
    return jnp.concatenate([pred_p[:_GHALF], pred_p[_GPAD:_GPAD + _GHALF]])
